# Initial kernel scaffold; baseline (speedup 1.0000x reference)
import jax
import jax.numpy as jnp
from jax.experimental import pallas as pl


def kernel(input_ids, embed_table):
    raise NotImplementedError("write your pallas kernel here")


# speedup vs baseline: 1.4756x; 1.4756x over previous
import functools

import jax
import jax.numpy as jnp
from jax import lax
from jax.experimental import pallas as pl
from jax.experimental.pallas import tpu as pltpu
from jax.experimental.pallas import tpu_sc as plsc

_HIDDEN = 32
_B = 4096 * 200
_NC = 2
_NS = 16
_NW = _NC * _NS
_BPW = _B // _NW
_CHUNK = 128
_NCH = _BPW // _CHUNK
_NBUF = 8
_NGRP = _NCH // _NBUF


def _emb_body(ids_hbm, table_hbm, out_hbm, idx_v, rows_v, gsem, osem):
    wid = lax.axis_index("s") * _NC + lax.axis_index("c")
    pltpu.sync_copy(ids_hbm.at[wid], idx_v)
    out_base = wid * _BPW

    def group(g, carry):
        j0 = g * _NBUF
        gathers = [
            pltpu.async_copy(table_hbm.at[idx_v.at[j0 + b]], rows_v.at[b], gsem)
            for b in range(_NBUF)
        ]
        for d in gathers:
            d.wait()
        stores = [
            pltpu.async_copy(
                rows_v.at[b],
                out_hbm.at[pl.ds(out_base + (j0 + b) * _CHUNK, _CHUNK)],
                osem,
            )
            for b in range(_NBUF)
        ]
        for d in stores:
            d.wait()
        return carry

    lax.fori_loop(0, _NGRP, group, 0)


@jax.jit
def _run(ids_flat, table):
    mesh = plsc.VectorSubcoreMesh(core_axis_name="c", subcore_axis_name="s")
    f = functools.partial(
        pl.kernel,
        mesh=mesh,
        compiler_params=pltpu.CompilerParams(use_tc_tiling_on_sc=False),
        out_type=jax.ShapeDtypeStruct((_B, _HIDDEN), jnp.float32),
        scratch_types=[
            pltpu.VMEM((_NCH, _CHUNK), jnp.int32),
            pltpu.VMEM((_NBUF, _CHUNK, _HIDDEN), jnp.float32),
            pltpu.SemaphoreType.DMA,
            pltpu.SemaphoreType.DMA,
        ],
    )(_emb_body)
    return f(ids_flat, table)


def kernel(input_ids, embed_table):
    batch, hist = input_ids.shape
    ids_flat = input_ids.reshape(_NW, _NCH, _CHUNK).astype(jnp.int32)
    out = _run(ids_flat, embed_table)
    return out.reshape(batch, hist, _HIDDEN)


# speedup vs baseline: 1.5002x; 1.0167x over previous
import functools

import jax
import jax.numpy as jnp
from jax import lax
from jax.experimental import pallas as pl
from jax.experimental.pallas import tpu as pltpu
from jax.experimental.pallas import tpu_sc as plsc

_HIDDEN = 32
_B = 4096 * 200
_NC = 2
_NS = 16
_NW = _NC * _NS
_BPW = _B // _NW
_CHUNK = 128
_NCH = _BPW // _CHUNK
_G = 20
_NGRP = _NCH // _G
_HALF = _G // 2


def _emb_body(ids_hbm, table_hbm, out_hbm, idx_v, rows_v, gsem, osem):
    wid = lax.axis_index("s") * _NC + lax.axis_index("c")
    pltpu.sync_copy(ids_hbm.at[wid], idx_v)
    out_base = wid * _BPW

    def group(g, carry):
        j0 = g * _G
        gathers = [
            pltpu.async_copy(
                table_hbm.at[idx_v.at[j0 + b]],
                rows_v.at[pl.ds(b * _CHUNK, _CHUNK)],
                gsem,
            )
            for b in range(_G)
        ]
        for b in range(_HALF):
            gathers[b].wait()
        s1 = pltpu.async_copy(
            rows_v.at[pl.ds(0, _HALF * _CHUNK)],
            out_hbm.at[pl.ds(out_base + j0 * _CHUNK, _HALF * _CHUNK)],
            osem,
        )
        for b in range(_HALF, _G):
            gathers[b].wait()
        s2 = pltpu.async_copy(
            rows_v.at[pl.ds(_HALF * _CHUNK, _HALF * _CHUNK)],
            out_hbm.at[pl.ds(out_base + (j0 + _HALF) * _CHUNK, _HALF * _CHUNK)],
            osem,
        )
        s1.wait()
        s2.wait()
        return carry

    lax.fori_loop(0, _NGRP, group, 0)


@jax.jit
def _run(ids_flat, table):
    mesh = plsc.VectorSubcoreMesh(core_axis_name="c", subcore_axis_name="s")
    f = functools.partial(
        pl.kernel,
        mesh=mesh,
        compiler_params=pltpu.CompilerParams(use_tc_tiling_on_sc=False),
        out_type=jax.ShapeDtypeStruct((_B, _HIDDEN), jnp.float32),
        scratch_types=[
            pltpu.VMEM((_NCH, _CHUNK), jnp.int32),
            pltpu.VMEM((_G * _CHUNK, _HIDDEN), jnp.float32),
            pltpu.SemaphoreType.DMA,
            pltpu.SemaphoreType.DMA,
        ],
    )(_emb_body)
    return f(ids_flat, table)


def kernel(input_ids, embed_table):
    batch, hist = input_ids.shape
    ids_flat = input_ids.reshape(_NW, _NCH, _CHUNK).astype(jnp.int32)
    out = _run(ids_flat, embed_table)
    return out.reshape(batch, hist, _HIDDEN)
